# baseline (speedup 1.0000x reference)
import functools

import jax
import jax.numpy as jnp
from jax import lax
from jax.experimental import pallas as pl
from jax.experimental.pallas import tpu as pltpu
from jax.experimental.pallas import tpu_sc as plsc

B, S, SW, D = 1024, 200, 20, 64
N = B * S
L = 16

IDX_COLS = 80
TOK_PER_BATCH = 40
ROWS_PER_BATCH = TOK_PER_BATCH * SW
GATHERS_PER_BATCH = ROWS_PER_BATCH // IDX_COLS
BATCH_PER_ROW = S // TOK_PER_BATCH


@functools.lru_cache(maxsize=None)
def _build_sc_kernel():
    info = plsc.get_sparse_core_info()
    nw = info.num_cores * info.num_subcores
    tok_per_w = N // nw
    rows_per_w = B // nw
    batches = tok_per_w // TOK_PER_BATCH

    mesh = plsc.VectorSubcoreMesh(core_axis_name="c", subcore_axis_name="s")

    @functools.partial(
        pl.kernel,
        mesh=mesh,
        out_type=jax.ShapeDtypeStruct((B, S, D), jnp.float32),
        compiler_params=pltpu.CompilerParams(use_tc_tiling_on_sc=False),
        scratch_types=[
            pltpu.VMEM((2, ROWS_PER_BATCH), jnp.int32),
            pltpu.VMEM((2, ROWS_PER_BATCH, D), jnp.float32),
            pltpu.VMEM((2, TOK_PER_BATCH, D), jnp.float32),
            pltpu.SemaphoreType.DMA((2,)),
            pltpu.SemaphoreType.DMA((2,)),
            pltpu.SemaphoreType.DMA((2,)),
        ],
    )
    def sc_kernel(table_hbm, idx_hbm, out_hbm, idx_v, rows_v, out_v,
                  sem_i, sem_g, sem_o):
        wid = lax.axis_index("s") * info.num_cores + lax.axis_index("c")
        idx0 = wid * tok_per_w * SW
        row0 = wid * rows_per_w

        def idx_copy(b, ph):
            return pltpu.make_async_copy(
                idx_hbm.at[pl.ds(idx0 + b * ROWS_PER_BATCH, ROWS_PER_BATCH)],
                idx_v.at[ph], sem_i.at[ph])

        def gather_copies(ph):
            return [
                pltpu.make_async_copy(
                    table_hbm.at[idx_v.at[ph, pl.ds(j * IDX_COLS, IDX_COLS)]],
                    rows_v.at[ph, pl.ds(j * IDX_COLS, IDX_COLS)],
                    sem_g.at[ph])
                for j in range(GATHERS_PER_BATCH)
            ]

        def out_copy(b, ph):
            srow = row0 + b // BATCH_PER_ROW
            scol = (b % BATCH_PER_ROW) * TOK_PER_BATCH
            return pltpu.make_async_copy(
                out_v.at[ph],
                out_hbm.at[srow, pl.ds(scol, TOK_PER_BATCH)],
                sem_o.at[ph])

        idx_copy(0, 0).start()
        idx_copy(0, 0).wait()
        for cp in gather_copies(0):
            cp.start()
        idx_copy(1, 1).start()

        def pair_body(g, carry):
            for ph in range(2):
                b = g * 2 + ph
                nxt = 1 - ph

                @pl.when(b + 1 < batches)
                def _():
                    idx_copy(b + 1, nxt).wait()
                    for cp in gather_copies(nxt):
                        cp.start()

                for cp in gather_copies(ph):
                    cp.wait()

                @pl.when(b + 2 < batches)
                def _():
                    idx_copy(b + 2, ph).start()

                @pl.when(b >= 2)
                def _():
                    out_copy(b - 2, ph).wait()

                def tok_body(t, c):
                    r0 = t * SW
                    for d in range(D // L):
                        sl = pl.ds(d * L, L)
                        acc = rows_v[ph, r0, sl]
                        for s in range(1, SW):
                            acc = acc + rows_v[ph, r0 + s, sl]
                        out_v[ph, t, sl] = acc * (1.0 / SW)
                    return c

                lax.fori_loop(0, TOK_PER_BATCH, tok_body, 0, unroll=False)

                out_copy(b, ph).start()
            return carry

        lax.fori_loop(0, batches // 2, pair_body, 0, unroll=False)

        out_copy(batches - 2, 0).wait()
        out_copy(batches - 1, 1).wait()

    return sc_kernel


def kernel(inpt, table):
    idx = inpt.reshape(N * SW).astype(jnp.int32)
    tbl = table.astype(jnp.float32)
    return _build_sc_kernel()(tbl, idx)


# speedup vs baseline: 1.0043x; 1.0043x over previous
import functools

import jax
import jax.numpy as jnp
from jax import lax
from jax.experimental import pallas as pl
from jax.experimental.pallas import tpu as pltpu
from jax.experimental.pallas import tpu_sc as plsc

B, S, SW, D = 1024, 200, 20, 64
N = B * S
L = 16
VOCAB2 = 1000002

IDX_COLS = 128
TOK_PER_BATCH = 32
ROWS_PER_BATCH = TOK_PER_BATCH * SW
GATHERS_PER_BATCH = ROWS_PER_BATCH // IDX_COLS
OUT_ROWS_PER_BATCH = TOK_PER_BATCH // 2


@functools.lru_cache(maxsize=None)
def _build_sc_kernel():
    info = plsc.get_sparse_core_info()
    nw = info.num_cores * info.num_subcores
    tok_per_w = N // nw
    batches = tok_per_w // TOK_PER_BATCH

    mesh = plsc.VectorSubcoreMesh(core_axis_name="c", subcore_axis_name="s")

    @functools.partial(
        pl.kernel,
        mesh=mesh,
        out_type=jax.ShapeDtypeStruct((N * D // 128, 128), jnp.float32),
        compiler_params=pltpu.CompilerParams(use_tc_tiling_on_sc=False),
        scratch_types=[
            pltpu.VMEM((2, GATHERS_PER_BATCH, IDX_COLS), jnp.int32),
            pltpu.VMEM((2, ROWS_PER_BATCH, D), jnp.float32),
            pltpu.VMEM((2, OUT_ROWS_PER_BATCH, 2 * D), jnp.float32),
            pltpu.SemaphoreType.DMA((2,)),
            pltpu.SemaphoreType.DMA((2,)),
            pltpu.SemaphoreType.DMA((2,)),
        ],
    )
    def sc_kernel(table_hbm, idx_hbm, out_hbm, idx_v, rows_v, out_v,
                  sem_i, sem_g, sem_o):
        wid = lax.axis_index("s") * info.num_cores + lax.axis_index("c")
        irow0 = wid * (tok_per_w * SW // IDX_COLS)
        orow0 = wid * (tok_per_w * D // 128)

        def idx_copy(b, ph):
            return pltpu.make_async_copy(
                idx_hbm.at[pl.ds(irow0 + b * GATHERS_PER_BATCH,
                                 GATHERS_PER_BATCH)],
                idx_v.at[ph], sem_i.at[ph])

        def gather_copies(ph):
            return [
                pltpu.make_async_copy(
                    table_hbm.at[idx_v.at[ph, j]],
                    rows_v.at[ph, pl.ds(j * IDX_COLS, IDX_COLS)],
                    sem_g.at[ph])
                for j in range(GATHERS_PER_BATCH)
            ]

        def out_copy(b, ph):
            return pltpu.make_async_copy(
                out_v.at[ph],
                out_hbm.at[pl.ds(orow0 + b * OUT_ROWS_PER_BATCH,
                                 OUT_ROWS_PER_BATCH)],
                sem_o.at[ph])

        idx_copy(0, 0).start()
        idx_copy(0, 0).wait()
        for cp in gather_copies(0):
            cp.start()
        idx_copy(1, 1).start()

        def pair_body(g, carry):
            for ph in range(2):
                b = g * 2 + ph
                nxt = 1 - ph

                @pl.when(b + 1 < batches)
                def _():
                    idx_copy(b + 1, nxt).wait()
                    for cp in gather_copies(nxt):
                        cp.start()

                for cp in gather_copies(ph):
                    cp.wait()

                @pl.when(b + 2 < batches)
                def _():
                    idx_copy(b + 2, ph).start()

                @pl.when(b >= 2)
                def _():
                    out_copy(b - 2, ph).wait()

                def pair_tok_body(p, c):
                    for par in range(2):
                        r0 = p * 2 * SW + par * SW
                        for d in range(D // L):
                            sl = pl.ds(par * D + d * L, L)
                            rsl = pl.ds(d * L, L)
                            acc = rows_v[ph, r0, rsl]
                            for s in range(1, SW):
                                acc = acc + rows_v[ph, r0 + s, rsl]
                            out_v[ph, p, sl] = acc * (1.0 / SW)
                    return c

                lax.fori_loop(0, OUT_ROWS_PER_BATCH, pair_tok_body, 0,
                              unroll=False)

                out_copy(b, ph).start()
            return carry

        lax.fori_loop(0, batches // 2, pair_body, 0, unroll=False)

        out_copy(batches - 2, 0).wait()
        out_copy(batches - 1, 1).wait()

    return sc_kernel


def kernel(inpt, table):
    idx = inpt.astype(jnp.int32).reshape(N * SW // 128, 128)
    tbl = table.astype(jnp.float32)
    out = _build_sc_kernel()(tbl, idx)
    return out.reshape(B, S, D)
